# baseline (speedup 1.0000x reference)
import functools

import jax
import jax.numpy as jnp
from jax.experimental import pallas as pl
from jax.experimental.pallas import tpu as pltpu

EMBED = 256
N = 8192
BM = 512
BN = 512
GI = N // BM
GJ = N // BN
NLANE = 128
NG = BN // NLANE
FBM = 1024


def _sweep_body(z_ref, w2x_ref, z2_ref, w2_ref, rmin_ref, rarg_ref):
    j = pl.program_id(1)

    @pl.when(j == 0)
    def _init():
        rmin_ref[...] = jnp.full((BM, NLANE), jnp.inf, jnp.float32)
        rarg_ref[...] = jnp.zeros((BM, NLANE), jnp.float32)

    zb = z_ref[...]
    wb = w2x_ref[...]
    mm2 = jax.lax.dot_general(
        zb, wb, (((1,), (1,)), ((), ())),
        preferred_element_type=jnp.float32)
    z2b = z2_ref[...]
    w2b = w2_ref[...]

    rm = rmin_ref[...]
    ra = rarg_ref[...]
    for g in range(NG):
        sl = slice(g * NLANE, (g + 1) * NLANE)
        dg = (z2b - mm2[:, sl]) + w2b[:, sl]
        sf = (j * NG + g).astype(jnp.float32)
        better = dg < rm
        rm = jnp.where(better, dg, rm)
        ra = jnp.where(better, sf, ra)
    rmin_ref[...] = rm
    rarg_ref[...] = ra


def _final_body(rmin_ref, rarg_ref, idx_ref):
    rm = rmin_ref[...]
    col = rarg_ref[...] * 128.0 + jax.lax.broadcasted_iota(
        jnp.int32, (FBM, NLANE), 1).astype(jnp.float32)
    gmin = jnp.min(rm, axis=1, keepdims=True)
    cand = jnp.where(rm == gmin, col, jnp.float32(2**24))
    idx_ref[...] = jnp.min(cand, axis=1, keepdims=True).astype(jnp.int32)


@functools.partial(jax.jit)
def _encode(z, W2x, z2, w2):
    rmin, rarg = pl.pallas_call(
        _sweep_body,
        grid=(GI, GJ),
        in_specs=[
            pl.BlockSpec((BM, EMBED), lambda i, j: (i, 0)),
            pl.BlockSpec((BN, EMBED), lambda i, j: (j, 0)),
            pl.BlockSpec((BM, 1), lambda i, j: (i, 0)),
            pl.BlockSpec((1, BN), lambda i, j: (0, j)),
        ],
        out_specs=[
            pl.BlockSpec((BM, NLANE), lambda i, j: (i, 0)),
            pl.BlockSpec((BM, NLANE), lambda i, j: (i, 0)),
        ],
        out_shape=[
            jax.ShapeDtypeStruct((N, NLANE), jnp.float32),
            jax.ShapeDtypeStruct((N, NLANE), jnp.float32),
        ],
        compiler_params=pltpu.CompilerParams(
            dimension_semantics=("parallel", "arbitrary")),
    )(z, W2x, z2, w2)
    idx = pl.pallas_call(
        _final_body,
        grid=(N // FBM,),
        in_specs=[
            pl.BlockSpec((FBM, NLANE), lambda i: (i, 0)),
            pl.BlockSpec((FBM, NLANE), lambda i: (i, 0)),
        ],
        out_specs=pl.BlockSpec((FBM, 1), lambda i: (i, 0)),
        out_shape=jax.ShapeDtypeStruct((N, 1), jnp.int32),
        compiler_params=pltpu.CompilerParams(
            dimension_semantics=("parallel",)),
    )(rmin, rarg)
    return idx


def kernel(z, W):
    z2 = jnp.sum(z ** 2, axis=1, keepdims=True)
    w2 = jnp.sum(W ** 2, axis=1)[None, :]
    idx = _encode(z, 2.0 * W, z2, w2)[:, 0]
    z_q = W[idx]
    commitment_loss = jnp.mean((jax.lax.stop_gradient(z_q) - z) ** 2)
    codebook_loss = jnp.mean((z_q - jax.lax.stop_gradient(z)) ** 2)
    vq_loss = codebook_loss + 0.25 * commitment_loss
    z_q_st = z + jax.lax.stop_gradient(z_q - z)
    return (z_q_st, vq_loss)


# speedup vs baseline: 1.4549x; 1.4549x over previous
import functools

import jax
import jax.numpy as jnp
from jax.experimental import pallas as pl
from jax.experimental.pallas import tpu as pltpu

EMBED = 256
N = 8192
BM = 1024
BN = 512
GI = N // BM
GJ = N // BN
NLANE = 128
NG = BN // NLANE
FBM = 1024


def _sweep_body(z_ref, w2x_ref, z2_ref, w2_ref, rmin_ref, rarg_ref,
                smin, sarg):
    j = pl.program_id(1)

    @pl.when(j == 0)
    def _init():
        smin[...] = jnp.full((BM, NLANE), jnp.inf, jnp.float32)
        sarg[...] = jnp.zeros((BM, NLANE), jnp.float32)

    zb = z_ref[...]
    wb = w2x_ref[...]
    mm2 = jax.lax.dot_general(
        zb, wb, (((1,), (1,)), ((), ())),
        preferred_element_type=jnp.float32)
    z2b = z2_ref[...]
    w2b = w2_ref[...]

    rm = smin[...]
    ra = sarg[...]
    for g in range(NG):
        sl = slice(g * NLANE, (g + 1) * NLANE)
        dg = (z2b - mm2[:, sl]) + w2b[:, sl]
        sf = (j * NG + g).astype(jnp.float32)
        better = dg < rm
        rm = jnp.where(better, dg, rm)
        ra = jnp.where(better, sf, ra)
    smin[...] = rm
    sarg[...] = ra

    @pl.when(j == GJ - 1)
    def _flush():
        rmin_ref[...] = rm
        rarg_ref[...] = ra


def _final_body(rmin_ref, rarg_ref, idx_ref):
    rm = rmin_ref[...]
    col = rarg_ref[...] * 128.0 + jax.lax.broadcasted_iota(
        jnp.int32, (FBM, NLANE), 1).astype(jnp.float32)
    gmin = jnp.min(rm, axis=1, keepdims=True)
    cand = jnp.where(rm == gmin, col, jnp.float32(2**24))
    idx_ref[...] = jnp.min(cand, axis=1, keepdims=True).astype(jnp.int32)


@functools.partial(jax.jit)
def _encode(z, W2x, z2, w2):
    rmin, rarg = pl.pallas_call(
        _sweep_body,
        grid=(GI, GJ),
        in_specs=[
            pl.BlockSpec((BM, EMBED), lambda i, j: (i, 0)),
            pl.BlockSpec((BN, EMBED), lambda i, j: (j, 0)),
            pl.BlockSpec((BM, 1), lambda i, j: (i, 0)),
            pl.BlockSpec((1, BN), lambda i, j: (0, j)),
        ],
        out_specs=[
            pl.BlockSpec((BM, NLANE), lambda i, j: (i, 0)),
            pl.BlockSpec((BM, NLANE), lambda i, j: (i, 0)),
        ],
        out_shape=[
            jax.ShapeDtypeStruct((N, NLANE), jnp.float32),
            jax.ShapeDtypeStruct((N, NLANE), jnp.float32),
        ],
        scratch_shapes=[
            pltpu.VMEM((BM, NLANE), jnp.float32),
            pltpu.VMEM((BM, NLANE), jnp.float32),
        ],
        compiler_params=pltpu.CompilerParams(
            dimension_semantics=("parallel", "arbitrary")),
    )(z, W2x, z2, w2)
    idx = pl.pallas_call(
        _final_body,
        grid=(N // FBM,),
        in_specs=[
            pl.BlockSpec((FBM, NLANE), lambda i: (i, 0)),
            pl.BlockSpec((FBM, NLANE), lambda i: (i, 0)),
        ],
        out_specs=pl.BlockSpec((FBM, 1), lambda i: (i, 0)),
        out_shape=jax.ShapeDtypeStruct((N, 1), jnp.int32),
        compiler_params=pltpu.CompilerParams(
            dimension_semantics=("parallel",)),
    )(rmin, rarg)
    return idx


def kernel(z, W):
    z2 = jnp.sum(z ** 2, axis=1, keepdims=True)
    w2 = jnp.sum(W ** 2, axis=1)[None, :]
    idx = _encode(z, 2.0 * W, z2, w2)[:, 0]
    z_q = W[idx]
    commitment_loss = jnp.mean((jax.lax.stop_gradient(z_q) - z) ** 2)
    codebook_loss = jnp.mean((z_q - jax.lax.stop_gradient(z)) ** 2)
    vq_loss = codebook_loss + 0.25 * commitment_loss
    z_q_st = z + jax.lax.stop_gradient(z_q - z)
    return (z_q_st, vq_loss)


# speedup vs baseline: 1.4855x; 1.0211x over previous
import functools

import jax
import jax.numpy as jnp
from jax.experimental import pallas as pl
from jax.experimental.pallas import tpu as pltpu

EMBED = 256
N = 8192
BM = 2048
BN = 512
GI = N // BM
GJ = N // BN
NLANE = 128
NG = BN // NLANE
FBM = 1024


def _sweep_body(z_ref, w_ref, z2_ref, w2_ref, rmin_ref, rarg_ref,
                mmbuf, smin, sarg):
    j = pl.program_id(1)

    @pl.when(j == 0)
    def _init():
        smin[...] = jnp.full((BM, NLANE), jnp.inf, jnp.float32)
        sarg[...] = jnp.zeros((BM, NLANE), jnp.float32)

    zb = z_ref[...]
    wb = w_ref[...] * 2.0
    mmbuf[j % 2] = jax.lax.dot_general(
        zb, wb, (((1,), (1,)), ((), ())),
        preferred_element_type=jnp.float32)

    @pl.when(j > 0)
    def _epilogue():
        mm2 = mmbuf[(j - 1) % 2]
        z2b = z2_ref[...]
        w2b = w2_ref[...]
        rm = smin[...]
        ra = sarg[...]
        for g in range(NG):
            sl = slice(g * NLANE, (g + 1) * NLANE)
            dg = (z2b - mm2[:, sl]) + w2b[:, sl]
            sf = ((j - 1) * NG + g).astype(jnp.float32)
            better = dg < rm
            rm = jnp.where(better, dg, rm)
            ra = jnp.where(better, sf, ra)
        smin[...] = rm
        sarg[...] = ra

    @pl.when(j == GJ)
    def _flush():
        rmin_ref[...] = smin[...]
        rarg_ref[...] = sarg[...]


def _final_body(rmin_ref, rarg_ref, idx_ref):
    rm = rmin_ref[...]
    col = rarg_ref[...] * 128.0 + jax.lax.broadcasted_iota(
        jnp.int32, (FBM, NLANE), 1).astype(jnp.float32)
    gmin = jnp.min(rm, axis=1, keepdims=True)
    cand = jnp.where(rm == gmin, col, jnp.float32(2**24))
    idx_ref[...] = jnp.min(cand, axis=1, keepdims=True).astype(jnp.int32)


@functools.partial(jax.jit)
def _encode(z, W, z2, w2):
    rmin, rarg = pl.pallas_call(
        _sweep_body,
        grid=(GI, GJ + 1),
        in_specs=[
            pl.BlockSpec((BM, EMBED), lambda i, j: (i, 0)),
            pl.BlockSpec((BN, EMBED), lambda i, j: (jnp.minimum(j, GJ - 1), 0)),
            pl.BlockSpec((BM, 1), lambda i, j: (i, 0)),
            pl.BlockSpec((1, BN), lambda i, j: (0, jnp.maximum(j - 1, 0))),
        ],
        out_specs=[
            pl.BlockSpec((BM, NLANE), lambda i, j: (i, 0)),
            pl.BlockSpec((BM, NLANE), lambda i, j: (i, 0)),
        ],
        out_shape=[
            jax.ShapeDtypeStruct((N, NLANE), jnp.float32),
            jax.ShapeDtypeStruct((N, NLANE), jnp.float32),
        ],
        scratch_shapes=[
            pltpu.VMEM((2, BM, BN), jnp.float32),
            pltpu.VMEM((BM, NLANE), jnp.float32),
            pltpu.VMEM((BM, NLANE), jnp.float32),
        ],
        compiler_params=pltpu.CompilerParams(
            dimension_semantics=("parallel", "arbitrary")),
    )(z, W, z2, w2)
    idx = pl.pallas_call(
        _final_body,
        grid=(N // FBM,),
        in_specs=[
            pl.BlockSpec((FBM, NLANE), lambda i: (i, 0)),
            pl.BlockSpec((FBM, NLANE), lambda i: (i, 0)),
        ],
        out_specs=pl.BlockSpec((FBM, 1), lambda i: (i, 0)),
        out_shape=jax.ShapeDtypeStruct((N, 1), jnp.int32),
        compiler_params=pltpu.CompilerParams(
            dimension_semantics=("parallel",)),
    )(rmin, rarg)
    return idx


def kernel(z, W):
    z2 = jnp.sum(z ** 2, axis=1, keepdims=True)
    w2 = jnp.sum(W ** 2, axis=1)[None, :]
    idx = _encode(z, W, z2, w2)[:, 0]
    z_q = W[idx]
    commitment_loss = jnp.mean((jax.lax.stop_gradient(z_q) - z) ** 2)
    codebook_loss = jnp.mean((z_q - jax.lax.stop_gradient(z)) ** 2)
    vq_loss = codebook_loss + 0.25 * commitment_loss
    z_q_st = z + jax.lax.stop_gradient(z_q - z)
    return (z_q_st, vq_loss)


# speedup vs baseline: 1.6515x; 1.1117x over previous
import functools

import jax
import jax.numpy as jnp
from jax.experimental import pallas as pl
from jax.experimental.pallas import tpu as pltpu

EMBED = 256
N = 8192
BM = 2048
BN = 512
GI = N // BM
GJ = N // BN
GT = GJ // 2 + 1
NLANE = 128
NG = BN // NLANE
FBM = 1024


def _epilogue(mm_ref, z2b, w2b, smin, sarg, jb):
    rm = smin[...]
    ra = sarg[...]
    for g in range(NG):
        sl = slice(g * NLANE, (g + 1) * NLANE)
        dg = (z2b - mm_ref[:, sl]) + w2b[:, sl]
        sf = (jb * NG + g).astype(jnp.float32)
        better = dg < rm
        rm = jnp.where(better, dg, rm)
        ra = jnp.where(better, sf, ra)
    smin[...] = rm
    sarg[...] = ra


def _sweep_body(z_ref, wa_ref, wb_ref, z2_ref, w2a_ref, w2b_ref,
                rmin_ref, rarg_ref, mmA, mmB, smin, sarg):
    t = pl.program_id(1)
    zb = z_ref[...]
    z2b = z2_ref[...]

    @pl.when(t == 0)
    def _init():
        smin[...] = jnp.full((BM, NLANE), jnp.inf, jnp.float32)
        sarg[...] = jnp.zeros((BM, NLANE), jnp.float32)

    @pl.when(t > 0)
    def _epi_prev():
        _epilogue(mmB, z2b, w2b_ref[...], smin, sarg, 2 * t - 1)

    mmA[...] = jax.lax.dot_general(
        zb, wa_ref[...] * 2.0, (((1,), (1,)), ((), ())),
        preferred_element_type=jnp.float32)

    _epilogue(mmA, z2b, w2a_ref[...], smin, sarg, 2 * t)

    mmB[...] = jax.lax.dot_general(
        zb, wb_ref[...] * 2.0, (((1,), (1,)), ((), ())),
        preferred_element_type=jnp.float32)

    @pl.when(t == GT - 1)
    def _flush():
        rmin_ref[...] = smin[...]
        rarg_ref[...] = sarg[...]


def _final_body(rmin_ref, rarg_ref, idx_ref):
    rm = rmin_ref[...]
    col = rarg_ref[...] * 128.0 + jax.lax.broadcasted_iota(
        jnp.int32, (FBM, NLANE), 1).astype(jnp.float32)
    gmin = jnp.min(rm, axis=1, keepdims=True)
    cand = jnp.where(rm == gmin, col, jnp.float32(2**24))
    idx_ref[...] = jnp.min(cand, axis=1, keepdims=True).astype(jnp.int32)


@functools.partial(jax.jit)
def _encode(z, W, z2, w2):
    clamp = GJ - 1
    rmin, rarg = pl.pallas_call(
        _sweep_body,
        grid=(GI, GT),
        in_specs=[
            pl.BlockSpec((BM, EMBED), lambda i, t: (i, 0)),
            pl.BlockSpec((BN, EMBED), lambda i, t: (jnp.minimum(2 * t, clamp), 0)),
            pl.BlockSpec((BN, EMBED), lambda i, t: (jnp.minimum(2 * t + 1, clamp), 0)),
            pl.BlockSpec((BM, 1), lambda i, t: (i, 0)),
            pl.BlockSpec((1, BN), lambda i, t: (0, jnp.minimum(2 * t, clamp))),
            pl.BlockSpec((1, BN), lambda i, t: (0, jnp.maximum(2 * t - 1, 0))),
        ],
        out_specs=[
            pl.BlockSpec((BM, NLANE), lambda i, t: (i, 0)),
            pl.BlockSpec((BM, NLANE), lambda i, t: (i, 0)),
        ],
        out_shape=[
            jax.ShapeDtypeStruct((N, NLANE), jnp.float32),
            jax.ShapeDtypeStruct((N, NLANE), jnp.float32),
        ],
        scratch_shapes=[
            pltpu.VMEM((BM, BN), jnp.float32),
            pltpu.VMEM((BM, BN), jnp.float32),
            pltpu.VMEM((BM, NLANE), jnp.float32),
            pltpu.VMEM((BM, NLANE), jnp.float32),
        ],
        compiler_params=pltpu.CompilerParams(
            dimension_semantics=("parallel", "arbitrary")),
    )(z, W, W, z2, w2, w2)
    idx = pl.pallas_call(
        _final_body,
        grid=(N // FBM,),
        in_specs=[
            pl.BlockSpec((FBM, NLANE), lambda i: (i, 0)),
            pl.BlockSpec((FBM, NLANE), lambda i: (i, 0)),
        ],
        out_specs=pl.BlockSpec((FBM, 1), lambda i: (i, 0)),
        out_shape=jax.ShapeDtypeStruct((N, 1), jnp.int32),
        compiler_params=pltpu.CompilerParams(
            dimension_semantics=("parallel",)),
    )(rmin, rarg)
    return idx


def kernel(z, W):
    z2 = jnp.sum(z ** 2, axis=1, keepdims=True)
    w2 = jnp.sum(W ** 2, axis=1)[None, :]
    idx = _encode(z, W, z2, w2)[:, 0]
    z_q = W[idx]
    commitment_loss = jnp.mean((jax.lax.stop_gradient(z_q) - z) ** 2)
    codebook_loss = jnp.mean((z_q - jax.lax.stop_gradient(z)) ** 2)
    vq_loss = codebook_loss + 0.25 * commitment_loss
    z_q_st = z + jax.lax.stop_gradient(z_q - z)
    return (z_q_st, vq_loss)
